# baseline (speedup 1.0000x reference)
import jax
import jax.numpy as jnp
from jax import lax
from jax.experimental import pallas as pl
from jax.experimental.pallas import tpu as pltpu
from jax.experimental.pallas import tpu_sc as plsc

B = 16384
D = 16
NC = 2
NS = 16
NW = NC * NS
CHUNK = B // NW
GROUP = 16
NGROUP = CHUNK // GROUP
NRING = 2

LOG_MIN = -13.815511
LN2 = 0.6931472
SQRT2 = 1.4142135


def _vlog(x):
    bits = lax.bitcast_convert_type(x, jnp.int32)
    e = lax.shift_right_arithmetic(bits, 23) - 127
    m = lax.bitcast_convert_type(
        (bits & 0x7FFFFF) | 0x3F800000, jnp.float32)
    big = m > SQRT2
    m = jnp.where(big, m * 0.5, m)
    e = e + jnp.where(big, 1, 0)
    s = (m - 1.0) / (m + 1.0)
    z = s * s
    p = 2.0 * s * (1.0 + z * (0.33333334 + z * (0.2 + z * 0.14285715)))
    return e.astype(jnp.float32) * LN2 + p


def _sc_body(mus_hbm, ann_hbm, vals_hbm, conf_hbm, dummy_hbm, tabT_hbm,
             out_w, out_t,
             idx_s, rows_v, xbuf, vals_v, conf_v, wout_v, mus_v,
             totbuf_v, tot8_v, big_v, shared, ring_v, sem0, sem1):
    sems = (sem0, sem1)
    cid = lax.axis_index("c")
    sid = lax.axis_index("s")
    gid = cid * NS + sid
    base = gid * CHUNK

    pltpu.sync_copy(ann_hbm.at[pl.ds(base, CHUNK)], idx_s.at[pl.ds(0, CHUNK)])
    pltpu.sync_copy(vals_hbm.at[pl.ds(base, CHUNK)], vals_v)
    pltpu.sync_copy(conf_hbm.at[pl.ds(base, CHUNK)], conf_v)
    pltpu.sync_copy(mus_hbm, mus_v)
    idx_s[pl.ds(CHUNK, GROUP)] = jnp.zeros((16,), jnp.int32)
    idx_s[pl.ds(CHUNK + GROUP, GROUP)] = jnp.zeros((16,), jnp.int32)

    emus = jnp.exp(mus_v[...])
    lane = lax.iota(jnp.int32, 16)

    def fire_group(g, ring):
        ids = idx_s[pl.ds(g * GROUP, GROUP)]
        for k in range(GROUP):
            r = ids[k]
            blk = pl.multiple_of((r // 128) * 128, 128)
            pltpu.async_copy(
                tabT_hbm.at[:, pl.ds(blk, 128)],
                ring_v.at[pl.ds((ring * GROUP + k) * D, D), :],
                sems[ring])

    def wait_group(ring):
        pltpu.make_async_copy(
            dummy_hbm, ring_v.at[pl.ds(ring * GROUP * D, GROUP * D), :],
            sems[ring]).wait()

    fire_group(0, 0)
    fire_group(1, 1)

    def process_group(g, ring, acc):
        wait_group(ring)
        rid = lane + g * D
        maxv = jnp.full((16,), -1e30, jnp.float32)
        cols = idx_s[pl.ds(g * GROUP, GROUP)] % 128
        for k in range(GROUP):
            col = cols[k]
            v = plsc.load_gather(
                ring_v.at[pl.ds((ring * GROUP + k) * D, D), :],
                [lane, jnp.full((16,), col, jnp.int32)])
            rows_v[pl.ds((g * GROUP + k) * D, D)] = v
        fire_group(g + NRING, ring)
        for j in range(D):
            xj = plsc.load_gather(rows_v, [rid * D + j]) + emus[j]
            xbuf[pl.ds(j * 16, 16)] = xj
            maxv = jnp.maximum(maxv, xj)
        sumv = jnp.zeros((16,), jnp.float32)
        for j in range(D):
            sumv = sumv + jnp.exp(xbuf[pl.ds(j * 16, 16)] - maxv)
        vals_g = vals_v[pl.ds(g * D, D)]
        logit_v = plsc.load_gather(xbuf, [vals_g * 16 + lane])
        ll = logit_v - maxv - _vlog(sumv)
        ll = jnp.maximum(ll, LOG_MIN)
        w = conf_v[pl.ds(g * D, D)] * ll
        wout_v[pl.ds(g * D, D)] = w
        return acc + w

    def stage_pair(h, acc):
        acc = process_group(h * NRING, 0, acc)
        acc = process_group(h * NRING + 1, 1, acc)
        return acc

    acc = lax.fori_loop(0, NGROUP // NRING, stage_pair,
                        jnp.zeros((16,), jnp.float32))
    wait_group(0)
    wait_group(1)

    pltpu.sync_copy(wout_v, out_w.at[pl.ds(base, CHUNK)])

    totbuf_v[...] = acc
    pltpu.sync_copy(totbuf_v, shared.at[pl.ds(sid * 16, 16)])
    plsc.subcore_barrier()

    @pl.when(sid == 0)
    def _():
        pltpu.sync_copy(shared, big_v)
        svec = jnp.zeros((16,), jnp.float32)
        for r in range(NS):
            svec = svec + big_v[pl.ds(r * 16, 16)]
        t = jnp.sum(svec)
        tvec = jnp.full((16,), t, jnp.float32)
        for p in range(8):
            tot8_v[p, :] = tvec
        pltpu.sync_copy(tot8_v, out_t.at[cid])


@jax.jit
def _run(mus, table, annotators, values, confidences):
    mesh = plsc.VectorSubcoreMesh(core_axis_name="c", subcore_axis_name="s")
    f = pl.kernel(
        _sc_body,
        out_type=(
            jax.ShapeDtypeStruct((B,), jnp.float32),
            jax.ShapeDtypeStruct((NC, 8, 16), jnp.float32),
        ),
        mesh=mesh,
        compiler_params=pltpu.CompilerParams(
            needs_layout_passes=False,
            use_tc_tiling_on_sc=True,
        ),
        scratch_types=[
            pltpu.VMEM((CHUNK + 2 * GROUP,), jnp.int32),
            pltpu.VMEM((CHUNK * D,), jnp.float32),
            pltpu.VMEM((D * 16,), jnp.float32),
            pltpu.VMEM((CHUNK,), jnp.int32),
            pltpu.VMEM((CHUNK,), jnp.float32),
            pltpu.VMEM((CHUNK,), jnp.float32),
            pltpu.VMEM((16,), jnp.float32),
            pltpu.VMEM((16,), jnp.float32),
            pltpu.VMEM((8, 16), jnp.float32),
            pltpu.VMEM((NS * 16,), jnp.float32),
            pltpu.VMEM_SHARED((NS * 16,), jnp.float32),
            pltpu.VMEM((NRING * GROUP * D, 128), jnp.float32),
            pltpu.SemaphoreType.DMA,
            pltpu.SemaphoreType.DMA,
        ],
    )
    dummy = jnp.zeros((GROUP * D, 128), jnp.float32)
    out_w, out_t = f(mus, annotators.astype(jnp.int32),
                     values.astype(jnp.int32), confidences, dummy, table.T)
    return out_w, out_t


def kernel(mus, table, annotators, values, confidences):
    out_w, out_t = _run(mus, table, annotators, values, confidences)
    total = out_t[0, 0, 0] + out_t[1, 0, 0]
    return (out_w, total)


# speedup vs baseline: 1.0813x; 1.0813x over previous
import jax
import jax.numpy as jnp
from jax import lax
from jax.experimental import pallas as pl
from jax.experimental.pallas import tpu as pltpu
from jax.experimental.pallas import tpu_sc as plsc

B = 16384
D = 16
NC = 2
NS = 16
NW = NC * NS
CHUNK = B // NW
GROUP = 16
NGROUP = CHUNK // GROUP
ROWSTRIDE = 17

LOG_MIN = -13.815511
LN2 = 0.6931472
SQRT2 = 1.4142135


def _vlog(x):
    bits = lax.bitcast_convert_type(x, jnp.int32)
    e = lax.shift_right_arithmetic(bits, 23) - 127
    m = lax.bitcast_convert_type(
        (bits & 0x7FFFFF) | 0x3F800000, jnp.float32)
    big = m > SQRT2
    m = jnp.where(big, m * 0.5, m)
    e = e + jnp.where(big, 1, 0)
    s = (m - 1.0) / (m + 1.0)
    z = s * s
    p = 2.0 * s * (1.0 + z * (0.33333334 + z * (0.2 + z * 0.14285715)))
    return e.astype(jnp.float32) * LN2 + p


def _sc_body(mus_hbm, ann_hbm, vals_hbm, conf_hbm, tabT_hbm,
             out_w, out_t,
             idx_v, rows_v, xbuf, vals_v, conf_v, wout_v, mus_v,
             totbuf_v, tot8_v, big_v, shared, *bufs_and_sems):
    bufs = bufs_and_sems[:GROUP]
    sems = bufs_and_sems[GROUP:]
    cid = lax.axis_index("c")
    sid = lax.axis_index("s")
    gid = cid * NS + sid
    base = gid * CHUNK

    pltpu.sync_copy(ann_hbm.at[pl.ds(base, CHUNK)], idx_v.at[pl.ds(0, CHUNK)])
    pltpu.sync_copy(vals_hbm.at[pl.ds(base, CHUNK)], vals_v)
    pltpu.sync_copy(conf_hbm.at[pl.ds(base, CHUNK)], conf_v)
    pltpu.sync_copy(mus_hbm, mus_v)
    idx_v[pl.ds(CHUNK, GROUP)] = jnp.zeros((16,), jnp.int32)

    emus = jnp.exp(mus_v[...])
    lane = lax.iota(jnp.int32, 16)

    def fire(ids_vec, k):
        r = ids_vec[k]
        blk = pl.multiple_of((r // 128) * 128, 128)
        pltpu.async_copy(tabT_hbm.at[:, pl.ds(blk, 128)], bufs[k], sems[k])

    ids0 = idx_v[pl.ds(0, GROUP)]
    for k in range(GROUP):
        fire(ids0, k)

    def stage_group(g, carry):
        ids_g, acc = carry
        ids_next = idx_v[pl.ds((g + 1) * GROUP, GROUP)]
        cols = ids_g % 128
        rid = lane + g * D
        maxv = jnp.full((16,), -1e30, jnp.float32)
        for k in range(GROUP):
            pltpu.make_async_copy(
                tabT_hbm.at[:, pl.ds(0, 128)], bufs[k], sems[k]).wait()
            v = plsc.load_gather(bufs[k], [lane, jnp.full((16,), cols[k],
                                                          jnp.int32)])
            plsc.store_scatter(
                rows_v, [jnp.full((16,), (g * GROUP + k) * ROWSTRIDE,
                                  jnp.int32) + lane], v)
            fire(ids_next, k)
        for j in range(D):
            xj = plsc.load_gather(rows_v, [rid * ROWSTRIDE + j]) + emus[j]
            xbuf[pl.ds(j * 16, 16)] = xj
            maxv = jnp.maximum(maxv, xj)
        sumv = jnp.zeros((16,), jnp.float32)
        for j in range(D):
            sumv = sumv + jnp.exp(xbuf[pl.ds(j * 16, 16)] - maxv)
        vals_g = vals_v[pl.ds(g * D, D)]
        logit_v = plsc.load_gather(xbuf, [vals_g * 16 + lane])
        ll = logit_v - maxv - _vlog(sumv)
        ll = jnp.maximum(ll, LOG_MIN)
        w = conf_v[pl.ds(g * D, D)] * ll
        wout_v[pl.ds(g * D, D)] = w
        return ids_next, acc + w

    _, acc = lax.fori_loop(0, NGROUP, stage_group,
                           (ids0, jnp.zeros((16,), jnp.float32)))
    for k in range(GROUP):
        pltpu.make_async_copy(
            tabT_hbm.at[:, pl.ds(0, 128)], bufs[k], sems[k]).wait()

    pltpu.sync_copy(wout_v, out_w.at[pl.ds(base, CHUNK)])

    totbuf_v[...] = acc
    pltpu.sync_copy(totbuf_v, shared.at[pl.ds(sid * 16, 16)])
    plsc.subcore_barrier()

    @pl.when(sid == 0)
    def _():
        pltpu.sync_copy(shared, big_v)
        svec = jnp.zeros((16,), jnp.float32)
        for r in range(NS):
            svec = svec + big_v[pl.ds(r * 16, 16)]
        t = jnp.sum(svec)
        tvec = jnp.full((16,), t, jnp.float32)
        for p in range(8):
            tot8_v[p, :] = tvec
        pltpu.sync_copy(tot8_v, out_t.at[cid])


@jax.jit
def _run(mus, table, annotators, values, confidences):
    mesh = plsc.VectorSubcoreMesh(core_axis_name="c", subcore_axis_name="s")
    f = pl.kernel(
        _sc_body,
        out_type=(
            jax.ShapeDtypeStruct((B,), jnp.float32),
            jax.ShapeDtypeStruct((NC, 8, 16), jnp.float32),
        ),
        mesh=mesh,
        compiler_params=pltpu.CompilerParams(
            needs_layout_passes=False,
            use_tc_tiling_on_sc=True,
        ),
        scratch_types=[
            pltpu.VMEM((CHUNK + GROUP,), jnp.int32),
            pltpu.VMEM((CHUNK * ROWSTRIDE,), jnp.float32),
            pltpu.VMEM((D * 16,), jnp.float32),
            pltpu.VMEM((CHUNK,), jnp.int32),
            pltpu.VMEM((CHUNK,), jnp.float32),
            pltpu.VMEM((CHUNK,), jnp.float32),
            pltpu.VMEM((16,), jnp.float32),
            pltpu.VMEM((16,), jnp.float32),
            pltpu.VMEM((8, 16), jnp.float32),
            pltpu.VMEM((NS * 16,), jnp.float32),
            pltpu.VMEM_SHARED((NS * 16,), jnp.float32),
        ] + [pltpu.VMEM((16, 128), jnp.float32)] * GROUP
          + [pltpu.SemaphoreType.DMA] * GROUP,
    )
    out_w, out_t = f(mus, annotators.astype(jnp.int32),
                     values.astype(jnp.int32), confidences, table.T)
    return out_w, out_t


def kernel(mus, table, annotators, values, confidences):
    out_w, out_t = _run(mus, table, annotators, values, confidences)
    total = out_t[0, 0, 0] + out_t[1, 0, 0]
    return (out_w, total)
